# baseline (speedup 1.0000x reference)
import dataclasses
import functools

import jax
import jax.numpy as jnp
from jax import lax
from jax.experimental import pallas as pl
from jax.experimental.pallas import tpu as pltpu
from jax.experimental.pallas import tpu_sc as plsc

MARGIN = 1.1
WEIGHT = 1.0
EPS = 1e-6
LOWE_RATIO_TH = 0.9

N = 2048
D = 512
BN = 256
NBLK = N // BN

NW = 32
EPW = N // NW
VEC = 16


def _row_sums(x2d):
    ones = jnp.ones((D, 1), jnp.float32)
    return lax.dot_general(x2d, ones, (((1,), (0,)), ((), ())),
                           preferred_element_type=jnp.float32)


def _norm_rows(x2d):
    n = jnp.sqrt(_row_sums(x2d * x2d))
    return x2d * (1.0 / jnp.maximum(n, 1e-12))


def _tc_kernel(nimg, q_ref, sf_ref, coli1_out, rowi_out, w_out,
               qn_s, a2_s, rowv_s, rowi_s):
    j = pl.program_id(0)

    @pl.when(j == 0)
    def _init():
        qn = _norm_rows(q_ref[...])
        qn_s[...] = qn
        a2_s[...] = _row_sums(qn * qn)
        rowv_s[...] = jnp.full((N, 1), jnp.inf, jnp.float32)
        rowi_s[...] = jnp.zeros((N, 1), jnp.int32)

    nneg = nimg - 2
    sfb = sf_ref[...]
    others = sfb[1:].reshape((nimg - 1) * BN, D)
    othersn = _norm_rows(others).reshape(nimg - 1, BN, D)
    pn = othersn[0]
    b2 = _row_sums(pn * pn)

    qn = qn_s[...]
    g = lax.dot_general(qn, pn, (((1,), (1,)), ((), ())),
                        preferred_element_type=jnp.float32)
    d2 = jnp.maximum(a2_s[...] + b2.reshape(1, BN) - 2.0 * g, 0.0)

    v1 = jnp.min(d2, axis=0)
    i1 = jnp.argmin(d2, axis=0).astype(jnp.int32)
    riota = lax.broadcasted_iota(jnp.int32, d2.shape, 0)
    masked = jnp.where(riota == i1[None, :], jnp.inf, d2)
    i2 = jnp.argmin(masked, axis=0).astype(jnp.int32)

    rv = jnp.min(d2, axis=1, keepdims=True)
    ri = jnp.argmin(d2, axis=1).astype(jnp.int32)[:, None] + j * BN
    take = rv < rowv_s[...]
    rowv_s[...] = jnp.where(take, rv, rowv_s[...])
    rowi_s[...] = jnp.where(take, ri, rowi_s[...])

    qbe = qn_s[pl.ds(j * BN, BN), :] + EPS
    dif = qbe - pn
    dm = jnp.sqrt(_row_sums(dif * dif))
    y = dm * dm
    for k in range(1, nneg + 1):
        dif = qbe - othersn[k]
        dm = jnp.sqrt(_row_sums(dif * dif))
        h = jnp.maximum(MARGIN - dm, 0.0)
        y = y + h * h
    y = y.reshape(BN)

    gidx = lax.iota(jnp.int32, BN) + j * BN
    ratio = jnp.sqrt(v1) / i2.astype(jnp.float32)
    colok = jnp.logical_and(i1 == gidx, ratio <= LOWE_RATIO_TH)

    coli1_out[j, :] = i1
    w_out[j, :] = jnp.where(colok, y, 0.0)

    @pl.when(j == NBLK - 1)
    def _final():
        rowi_out[...] = rowi_s[...].reshape(NBLK, BN)


def _sc_finalize_body(coli1_hbm, rowi_hbm, w_hbm, out_hbm,
                      rowi_v, coli1_v, w_v, acc_v):
    wid = lax.axis_index("s") * 2 + lax.axis_index("c")
    base = wid * EPW
    pltpu.sync_copy(rowi_hbm, rowi_v)
    pltpu.sync_copy(coli1_hbm.at[pl.ds(base, EPW)], coli1_v)
    pltpu.sync_copy(w_hbm.at[pl.ds(base, EPW)], w_v)
    acc = jnp.zeros((VEC,), jnp.float32)
    for v in range(EPW // VEC):
        idx = coli1_v[pl.ds(v * VEC, VEC)]
        r = plsc.load_gather(rowi_v, [idx])
        jvec = lax.iota(jnp.int32, VEC) + (base + v * VEC)
        wv = w_v[pl.ds(v * VEC, VEC)]
        acc = acc + jnp.where(r == jvec, wv, jnp.float32(0.0))
    acc_v[...] = acc
    pltpu.sync_copy(acc_v, out_hbm.at[wid])


@functools.cache
def _sc_finalize():
    mesh = plsc.VectorSubcoreMesh(core_axis_name="c", subcore_axis_name="s")
    cp = pltpu.CompilerParams()
    if "needs_layout_passes" in pltpu.CompilerParams.__dataclass_fields__:
        cp = dataclasses.replace(cp, needs_layout_passes=False)
    return pl.kernel(
        _sc_finalize_body,
        out_type=jax.ShapeDtypeStruct((NW, VEC), jnp.float32),
        mesh=mesh,
        compiler_params=cp,
        scratch_types=[
            pltpu.VMEM((N,), jnp.int32),
            pltpu.VMEM((EPW,), jnp.int32),
            pltpu.VMEM((EPW,), jnp.float32),
            pltpu.VMEM((VEC,), jnp.float32),
        ],
    )


@jax.jit
def kernel(superfeatures, target):
    del target
    nimg = superfeatures.shape[0]
    q = superfeatures[0]
    coli1, rowi, w = pl.pallas_call(
        functools.partial(_tc_kernel, nimg),
        grid=(NBLK,),
        in_specs=[
            pl.BlockSpec((N, D), lambda j: (0, 0)),
            pl.BlockSpec((nimg, BN, D), lambda j: (0, j, 0)),
        ],
        out_specs=[
            pl.BlockSpec((NBLK, BN), lambda j: (0, 0)),
            pl.BlockSpec((NBLK, BN), lambda j: (0, 0)),
            pl.BlockSpec((NBLK, BN), lambda j: (0, 0)),
        ],
        out_shape=[
            jax.ShapeDtypeStruct((NBLK, BN), jnp.int32),
            jax.ShapeDtypeStruct((NBLK, BN), jnp.int32),
            jax.ShapeDtypeStruct((NBLK, BN), jnp.float32),
        ],
        scratch_shapes=[
            pltpu.VMEM((N, D), jnp.float32),
            pltpu.VMEM((N, 1), jnp.float32),
            pltpu.VMEM((N, 1), jnp.float32),
            pltpu.VMEM((N, 1), jnp.int32),
        ],
    )(q, superfeatures)
    partials = _sc_finalize()(coli1.reshape(N), rowi.reshape(N), w.reshape(N))
    return 0.5 * WEIGHT * jnp.sum(partials)


# speedup vs baseline: 1.2905x; 1.2905x over previous
import dataclasses
import functools

import jax
import jax.numpy as jnp
from jax import lax
from jax.experimental import pallas as pl
from jax.experimental.pallas import tpu as pltpu
from jax.experimental.pallas import tpu_sc as plsc

MARGIN = 1.1
WEIGHT = 1.0
EPS = 1e-6
LOWE_RATIO_TH = 0.9

N = 2048
D = 512
BN = 256
NBLK = N // BN

NW = 32
EPW = N // NW
VEC = 16


def _row_sums(x2d):
    return jnp.sum(x2d, axis=-1, keepdims=True)


def _norm_rows(x2d):
    n = jnp.sqrt(_row_sums(x2d * x2d))
    return x2d * (1.0 / jnp.maximum(n, 1e-12))


def _tc_kernel(nimg, q_ref, sf_ref, coli1_out, rowi_out, w_out,
               qn_s, a2_s, rowv_s, rowi_s):
    j = pl.program_id(0)

    @pl.when(j == 0)
    def _init():
        qn = _norm_rows(q_ref[...])
        qn_s[...] = qn
        a2_s[...] = _row_sums(qn * qn)
        rowv_s[...] = jnp.full((N, 1), jnp.inf, jnp.float32)
        rowi_s[...] = jnp.zeros((N, 1), jnp.int32)

    nneg = nimg - 2
    sfb = sf_ref[...]
    others = sfb[1:].reshape((nimg - 1) * BN, D)
    othersn = _norm_rows(others).reshape(nimg - 1, BN, D)
    pn = othersn[0]
    b2 = _row_sums(pn * pn)

    qn = qn_s[...]
    g = lax.dot_general(qn, pn, (((1,), (1,)), ((), ())),
                        preferred_element_type=jnp.float32)
    d2 = jnp.maximum(a2_s[...] + b2.reshape(1, BN) - 2.0 * g, 0.0)

    v1 = jnp.min(d2, axis=0)
    i1 = jnp.argmin(d2, axis=0).astype(jnp.int32)
    riota = lax.broadcasted_iota(jnp.int32, d2.shape, 0)
    masked = jnp.where(riota == i1[None, :], jnp.inf, d2)
    i2 = jnp.argmin(masked, axis=0).astype(jnp.int32)

    rv = jnp.min(d2, axis=1, keepdims=True)
    ri = jnp.argmin(d2, axis=1).astype(jnp.int32)[:, None] + j * BN
    take = rv < rowv_s[...]
    rowv_s[...] = jnp.where(take, rv, rowv_s[...])
    rowi_s[...] = jnp.where(take, ri, rowi_s[...])

    qbe = qn_s[pl.ds(j * BN, BN), :] + EPS
    dif = qbe - pn
    dm = jnp.sqrt(_row_sums(dif * dif))
    y = dm * dm
    for k in range(1, nneg + 1):
        dif = qbe - othersn[k]
        dm = jnp.sqrt(_row_sums(dif * dif))
        h = jnp.maximum(MARGIN - dm, 0.0)
        y = y + h * h
    y = y.reshape(BN)

    gidx = lax.iota(jnp.int32, BN) + j * BN
    ratio = jnp.sqrt(v1) / i2.astype(jnp.float32)
    colok = jnp.logical_and(i1 == gidx, ratio <= LOWE_RATIO_TH)

    coli1_out[j, :] = i1
    w_out[j, :] = jnp.where(colok, y, 0.0)

    @pl.when(j == NBLK - 1)
    def _final():
        rowi_out[...] = rowi_s[...].reshape(NBLK, BN)


def _sc_finalize_body(coli1_hbm, rowi_hbm, w_hbm, out_hbm,
                      rowi_v, coli1_v, w_v, acc_v):
    wid = lax.axis_index("s") * 2 + lax.axis_index("c")
    base = wid * EPW
    pltpu.sync_copy(rowi_hbm, rowi_v)
    pltpu.sync_copy(coli1_hbm.at[pl.ds(base, EPW)], coli1_v)
    pltpu.sync_copy(w_hbm.at[pl.ds(base, EPW)], w_v)
    acc = jnp.zeros((VEC,), jnp.float32)
    for v in range(EPW // VEC):
        idx = coli1_v[pl.ds(v * VEC, VEC)]
        r = plsc.load_gather(rowi_v, [idx])
        jvec = lax.iota(jnp.int32, VEC) + (base + v * VEC)
        wv = w_v[pl.ds(v * VEC, VEC)]
        acc = acc + jnp.where(r == jvec, wv, jnp.float32(0.0))
    acc_v[...] = acc
    pltpu.sync_copy(acc_v, out_hbm.at[wid])


@functools.cache
def _sc_finalize():
    mesh = plsc.VectorSubcoreMesh(core_axis_name="c", subcore_axis_name="s")
    cp = pltpu.CompilerParams()
    if "needs_layout_passes" in pltpu.CompilerParams.__dataclass_fields__:
        cp = dataclasses.replace(cp, needs_layout_passes=False)
    return pl.kernel(
        _sc_finalize_body,
        out_type=jax.ShapeDtypeStruct((NW, VEC), jnp.float32),
        mesh=mesh,
        compiler_params=cp,
        scratch_types=[
            pltpu.VMEM((N,), jnp.int32),
            pltpu.VMEM((EPW,), jnp.int32),
            pltpu.VMEM((EPW,), jnp.float32),
            pltpu.VMEM((VEC,), jnp.float32),
        ],
    )


@jax.jit
def kernel(superfeatures, target):
    del target
    nimg = superfeatures.shape[0]
    q = superfeatures[0]
    coli1, rowi, w = pl.pallas_call(
        functools.partial(_tc_kernel, nimg),
        grid=(NBLK,),
        in_specs=[
            pl.BlockSpec((N, D), lambda j: (0, 0)),
            pl.BlockSpec((nimg, BN, D), lambda j: (0, j, 0)),
        ],
        out_specs=[
            pl.BlockSpec((NBLK, BN), lambda j: (0, 0)),
            pl.BlockSpec((NBLK, BN), lambda j: (0, 0)),
            pl.BlockSpec((NBLK, BN), lambda j: (0, 0)),
        ],
        out_shape=[
            jax.ShapeDtypeStruct((NBLK, BN), jnp.int32),
            jax.ShapeDtypeStruct((NBLK, BN), jnp.int32),
            jax.ShapeDtypeStruct((NBLK, BN), jnp.float32),
        ],
        scratch_shapes=[
            pltpu.VMEM((N, D), jnp.float32),
            pltpu.VMEM((N, 1), jnp.float32),
            pltpu.VMEM((N, 1), jnp.float32),
            pltpu.VMEM((N, 1), jnp.int32),
        ],
    )(q, superfeatures)
    partials = _sc_finalize()(coli1.reshape(N), rowi.reshape(N), w.reshape(N))
    return 0.5 * WEIGHT * jnp.sum(partials)


# speedup vs baseline: 1.3329x; 1.0329x over previous
import dataclasses
import functools

import jax
import jax.numpy as jnp
from jax import lax
from jax.experimental import pallas as pl
from jax.experimental.pallas import tpu as pltpu
from jax.experimental.pallas import tpu_sc as plsc

MARGIN = 1.1
WEIGHT = 1.0
EPS = 1e-6
LOWE_RATIO_TH = 0.9

N = 2048
D = 512
BN = 256
NBLK = N // BN

NW = 16
EPW = N // NW
VEC = 16


def _row_sums(x2d):
    return jnp.sum(x2d, axis=-1, keepdims=True)


def _norm_rows(x2d):
    n = jnp.sqrt(_row_sums(x2d * x2d))
    return x2d * (1.0 / jnp.maximum(n, 1e-12))


def _tc_kernel(nimg, q_ref, sf_ref, coli1_out, rowi_out, w_out,
               qn_s, a2_s, rowv_s, rowi_s):
    j = pl.program_id(0)

    @pl.when(j == 0)
    def _init():
        qn = _norm_rows(q_ref[...])
        qn_s[...] = qn
        a2_s[...] = _row_sums(qn * qn)
        rowv_s[...] = jnp.full((N, 1), jnp.inf, jnp.float32)
        rowi_s[...] = jnp.zeros((N, 1), jnp.int32)

    nneg = nimg - 2
    sfb = sf_ref[...]
    others = sfb[1:].reshape((nimg - 1) * BN, D)
    othersn = _norm_rows(others).reshape(nimg - 1, BN, D)
    pn = othersn[0]
    b2 = _row_sums(pn * pn)

    qn = qn_s[...]
    g = lax.dot_general(qn, pn, (((1,), (1,)), ((), ())),
                        preferred_element_type=jnp.float32)
    d2 = jnp.maximum(a2_s[...] + b2.reshape(1, BN) - 2.0 * g, 0.0)

    v1 = jnp.min(d2, axis=0)
    i1 = jnp.argmin(d2, axis=0).astype(jnp.int32)
    riota = lax.broadcasted_iota(jnp.int32, d2.shape, 0)
    masked = jnp.where(riota == i1[None, :], jnp.inf, d2)
    i2 = jnp.argmin(masked, axis=0).astype(jnp.int32)

    rv = jnp.min(d2, axis=1, keepdims=True)
    ri = jnp.argmin(d2, axis=1).astype(jnp.int32)[:, None] + j * BN
    take = rv < rowv_s[...]
    rowv_s[...] = jnp.where(take, rv, rowv_s[...])
    rowi_s[...] = jnp.where(take, ri, rowi_s[...])

    qbe = qn_s[pl.ds(j * BN, BN), :] + EPS
    dif = qbe - pn
    dm = jnp.sqrt(_row_sums(dif * dif))
    y = dm * dm
    for k in range(1, nneg + 1):
        dif = qbe - othersn[k]
        dm = jnp.sqrt(_row_sums(dif * dif))
        h = jnp.maximum(MARGIN - dm, 0.0)
        y = y + h * h
    y = y.reshape(BN)

    gidx = lax.iota(jnp.int32, BN) + j * BN
    ratio = jnp.sqrt(v1) / i2.astype(jnp.float32)
    colok = jnp.logical_and(i1 == gidx, ratio <= LOWE_RATIO_TH)

    coli1_out[j, :] = i1
    w_out[j, :] = jnp.where(colok, y, 0.0)

    @pl.when(j == NBLK - 1)
    def _final():
        rowi_out[...] = rowi_s[...].reshape(NBLK, BN)


def _sc_finalize_body(coli1_hbm, rowi_hbm, w_hbm, out_hbm,
                      rowi_v, coli1_v, w_v, acc_v):
    wid = lax.axis_index("s")
    base = wid * EPW
    pltpu.sync_copy(rowi_hbm, rowi_v)
    pltpu.sync_copy(coli1_hbm.at[pl.ds(base, EPW)], coli1_v)
    pltpu.sync_copy(w_hbm.at[pl.ds(base, EPW)], w_v)
    acc = jnp.zeros((VEC,), jnp.float32)
    for v in range(EPW // VEC):
        idx = coli1_v[pl.ds(v * VEC, VEC)]
        r = plsc.load_gather(rowi_v, [idx])
        jvec = lax.iota(jnp.int32, VEC) + (base + v * VEC)
        wv = w_v[pl.ds(v * VEC, VEC)]
        acc = acc + jnp.where(r == jvec, wv, jnp.float32(0.0))
    acc_v[...] = acc
    pltpu.sync_copy(acc_v, out_hbm.at[wid])


@functools.cache
def _sc_finalize():
    mesh = plsc.VectorSubcoreMesh(
        core_axis_name="c", subcore_axis_name="s", num_cores=1)
    cp = pltpu.CompilerParams()
    if "needs_layout_passes" in pltpu.CompilerParams.__dataclass_fields__:
        cp = dataclasses.replace(cp, needs_layout_passes=False)
    return pl.kernel(
        _sc_finalize_body,
        out_type=jax.ShapeDtypeStruct((NW, VEC), jnp.float32),
        mesh=mesh,
        compiler_params=cp,
        scratch_types=[
            pltpu.VMEM((N,), jnp.int32),
            pltpu.VMEM((EPW,), jnp.int32),
            pltpu.VMEM((EPW,), jnp.float32),
            pltpu.VMEM((VEC,), jnp.float32),
        ],
    )


@jax.jit
def kernel(superfeatures, target):
    del target
    nimg = superfeatures.shape[0]
    q = superfeatures[0]
    coli1, rowi, w = pl.pallas_call(
        functools.partial(_tc_kernel, nimg),
        grid=(NBLK,),
        in_specs=[
            pl.BlockSpec((N, D), lambda j: (0, 0)),
            pl.BlockSpec((nimg, BN, D), lambda j: (0, j, 0)),
        ],
        out_specs=[
            pl.BlockSpec((NBLK, BN), lambda j: (0, 0)),
            pl.BlockSpec((NBLK, BN), lambda j: (0, 0)),
            pl.BlockSpec((NBLK, BN), lambda j: (0, 0)),
        ],
        out_shape=[
            jax.ShapeDtypeStruct((NBLK, BN), jnp.int32),
            jax.ShapeDtypeStruct((NBLK, BN), jnp.int32),
            jax.ShapeDtypeStruct((NBLK, BN), jnp.float32),
        ],
        scratch_shapes=[
            pltpu.VMEM((N, D), jnp.float32),
            pltpu.VMEM((N, 1), jnp.float32),
            pltpu.VMEM((N, 1), jnp.float32),
            pltpu.VMEM((N, 1), jnp.int32),
        ],
    )(q, superfeatures)
    partials = _sc_finalize()(coli1.reshape(N), rowi.reshape(N), w.reshape(N))
    return 0.5 * WEIGHT * jnp.sum(partials)


# speedup vs baseline: 1.4326x; 1.0748x over previous
import dataclasses
import functools

import jax
import jax.numpy as jnp
from jax import lax
from jax.experimental import pallas as pl
from jax.experimental.pallas import tpu as pltpu
from jax.experimental.pallas import tpu_sc as plsc

MARGIN = 1.1
WEIGHT = 1.0
EPS = 1e-6
LOWE_RATIO_TH = 0.9

N = 2048
D = 512
BN = 256
NBLK = N // BN

NW = 16
EPW = N // NW
VEC = 16


def _row_sums(x2d):
    return jnp.sum(x2d, axis=-1, keepdims=True)


def _norm_rows(x2d):
    n = jnp.sqrt(_row_sums(x2d * x2d))
    return x2d * (1.0 / jnp.maximum(n, 1e-12))


def _tc_kernel(nimg, q_ref, sf_ref, coli1_out, rowi_out, w_out,
               qn_s, a2_s, a2t_s, rowv_s, rowi_s):
    j = pl.program_id(0)

    @pl.when(j == 0)
    def _init():
        qn = _norm_rows(q_ref[...])
        qn_s[...] = qn
        a2 = _row_sums(qn * qn)
        a2_s[...] = a2
        a2t_s[...] = a2.reshape(1, N)
        rowv_s[...] = jnp.full((1, N), jnp.inf, jnp.float32)
        rowi_s[...] = jnp.zeros((1, N), jnp.int32)

    nneg = nimg - 2
    sfb = sf_ref[...]
    others = sfb[1:].reshape((nimg - 1) * BN, D)
    othersn = _norm_rows(others).reshape(nimg - 1, BN, D)
    pn = othersn[0]
    b2 = _row_sums(pn * pn)

    qn = qn_s[...]
    g = lax.dot_general(qn, pn, (((1,), (1,)), ((), ())),
                        preferred_element_type=jnp.float32)
    d2 = jnp.maximum(a2_s[...] + b2.reshape(1, BN) - 2.0 * g, 0.0)

    v1 = jnp.min(d2, axis=0)
    i1 = jnp.argmin(d2, axis=0).astype(jnp.int32)
    riota = lax.broadcasted_iota(jnp.int32, d2.shape, 0)
    masked = jnp.where(riota == i1[None, :], jnp.inf, d2)
    i2 = jnp.argmin(masked, axis=0).astype(jnp.int32)

    gt = lax.dot_general(pn, qn, (((1,), (1,)), ((), ())),
                         preferred_element_type=jnp.float32)
    d2t = jnp.maximum(b2 + a2t_s[...] - 2.0 * gt, 0.0)
    rv = jnp.min(d2t, axis=0, keepdims=True)
    ri = (jnp.argmin(d2t, axis=0).astype(jnp.int32)
          .reshape(1, N) + j * BN)
    take = rv < rowv_s[...]
    rowv_s[...] = jnp.where(take, rv, rowv_s[...])
    rowi_s[...] = jnp.where(take, ri, rowi_s[...])

    qbe = qn_s[pl.ds(j * BN, BN), :] + EPS
    dif = qbe - pn
    dm = jnp.sqrt(_row_sums(dif * dif))
    y = dm * dm
    for k in range(1, nneg + 1):
        dif = qbe - othersn[k]
        dm = jnp.sqrt(_row_sums(dif * dif))
        h = jnp.maximum(MARGIN - dm, 0.0)
        y = y + h * h
    y = y.reshape(BN)

    gidx = lax.iota(jnp.int32, BN) + j * BN
    ratio = jnp.sqrt(v1) / i2.astype(jnp.float32)
    colok = jnp.logical_and(i1 == gidx, ratio <= LOWE_RATIO_TH)

    coli1_out[j, :] = i1
    w_out[j, :] = jnp.where(colok, y, 0.0)

    @pl.when(j == NBLK - 1)
    def _final():
        rowi_out[...] = rowi_s[...]


def _sc_finalize_body(coli1_hbm, rowi_hbm, w_hbm, out_hbm,
                      rowi_v, coli1_v, w_v, acc_v):
    wid = lax.axis_index("s")
    base = wid * EPW
    pltpu.sync_copy(rowi_hbm, rowi_v)
    pltpu.sync_copy(coli1_hbm.at[pl.ds(base, EPW)], coli1_v)
    pltpu.sync_copy(w_hbm.at[pl.ds(base, EPW)], w_v)
    acc = jnp.zeros((VEC,), jnp.float32)
    for v in range(EPW // VEC):
        idx = coli1_v[pl.ds(v * VEC, VEC)]
        r = plsc.load_gather(rowi_v, [idx])
        jvec = lax.iota(jnp.int32, VEC) + (base + v * VEC)
        wv = w_v[pl.ds(v * VEC, VEC)]
        acc = acc + jnp.where(r == jvec, wv, jnp.float32(0.0))
    acc_v[...] = acc
    pltpu.sync_copy(acc_v, out_hbm.at[wid])


@functools.cache
def _sc_finalize():
    mesh = plsc.VectorSubcoreMesh(
        core_axis_name="c", subcore_axis_name="s", num_cores=1)
    cp = pltpu.CompilerParams()
    if "needs_layout_passes" in pltpu.CompilerParams.__dataclass_fields__:
        cp = dataclasses.replace(cp, needs_layout_passes=False)
    return pl.kernel(
        _sc_finalize_body,
        out_type=jax.ShapeDtypeStruct((NW, VEC), jnp.float32),
        mesh=mesh,
        compiler_params=cp,
        scratch_types=[
            pltpu.VMEM((N,), jnp.int32),
            pltpu.VMEM((EPW,), jnp.int32),
            pltpu.VMEM((EPW,), jnp.float32),
            pltpu.VMEM((VEC,), jnp.float32),
        ],
    )


@jax.jit
def kernel(superfeatures, target):
    del target
    nimg = superfeatures.shape[0]
    q = superfeatures[0]
    coli1, rowi, w = pl.pallas_call(
        functools.partial(_tc_kernel, nimg),
        grid=(NBLK,),
        in_specs=[
            pl.BlockSpec((N, D), lambda j: (0, 0)),
            pl.BlockSpec((nimg, BN, D), lambda j: (0, j, 0)),
        ],
        out_specs=[
            pl.BlockSpec((NBLK, BN), lambda j: (0, 0)),
            pl.BlockSpec((1, N), lambda j: (0, 0)),
            pl.BlockSpec((NBLK, BN), lambda j: (0, 0)),
        ],
        out_shape=[
            jax.ShapeDtypeStruct((NBLK, BN), jnp.int32),
            jax.ShapeDtypeStruct((1, N), jnp.int32),
            jax.ShapeDtypeStruct((NBLK, BN), jnp.float32),
        ],
        scratch_shapes=[
            pltpu.VMEM((N, D), jnp.float32),
            pltpu.VMEM((N, 1), jnp.float32),
            pltpu.VMEM((1, N), jnp.float32),
            pltpu.VMEM((1, N), jnp.float32),
            pltpu.VMEM((1, N), jnp.int32),
        ],
    )(q, superfeatures)
    partials = _sc_finalize()(coli1.reshape(N), rowi.reshape(N), w.reshape(N))
    return 0.5 * WEIGHT * jnp.sum(partials)


# speedup vs baseline: 1.6469x; 1.1496x over previous
import dataclasses
import functools

import jax
import jax.numpy as jnp
from jax import lax
from jax.experimental import pallas as pl
from jax.experimental.pallas import tpu as pltpu
from jax.experimental.pallas import tpu_sc as plsc

MARGIN = 1.1
WEIGHT = 1.0
EPS = 1e-6
LOWE_RATIO_TH = 0.9

N = 2048
D = 512
BN = 256
NBLK = N // BN

NW = 16
EPW = N // NW
VEC = 16


def _row_sums(x2d):
    return jnp.sum(x2d, axis=-1, keepdims=True)


def _norm_rows(x2d):
    n = jnp.sqrt(_row_sums(x2d * x2d))
    return x2d * (1.0 / jnp.maximum(n, 1e-12))


def _tc_kernel(nimg, q_ref, sf_ref, coli1_out, rowi_out, w_out,
               qn_s, qm2_s, a2_s, a2t_s, rowv_s, rowi_s):
    j = pl.program_id(0)

    @pl.when(j == 0)
    def _init():
        qn = _norm_rows(q_ref[0])
        qn_s[...] = qn
        qm2_s[...] = -2.0 * qn
        a2 = _row_sums(qn * qn)
        a2_s[...] = a2
        a2t_s[...] = a2.reshape(1, N)
        rowv_s[...] = jnp.full((1, N), jnp.inf, jnp.float32)
        rowi_s[...] = jnp.zeros((1, N), jnp.int32)

    nneg = nimg - 2
    sfb = sf_ref[...]
    others = sfb[1:].reshape((nimg - 1) * BN, D)
    othersn = _norm_rows(others).reshape(nimg - 1, BN, D)
    pn = othersn[0]
    b2 = _row_sums(pn * pn)

    qm2 = qm2_s[...]
    gm2 = lax.dot_general(qm2, pn, (((1,), (1,)), ((), ())),
                          preferred_element_type=jnp.float32)
    d2 = jnp.maximum(a2_s[...] + b2.reshape(1, BN) + gm2, 0.0)

    v1 = jnp.min(d2, axis=0)
    i1 = jnp.argmin(d2, axis=0).astype(jnp.int32)
    riota = lax.broadcasted_iota(jnp.int32, d2.shape, 0)
    masked = jnp.where(riota == i1[None, :], jnp.inf, d2)
    i2 = jnp.argmin(masked, axis=0).astype(jnp.int32)

    gtm2 = lax.dot_general(pn, qm2, (((1,), (1,)), ((), ())),
                           preferred_element_type=jnp.float32)
    d2t = jnp.maximum(b2 + a2t_s[...] + gtm2, 0.0)
    rv = jnp.min(d2t, axis=0, keepdims=True)
    ri = (jnp.argmin(d2t, axis=0).astype(jnp.int32)
          .reshape(1, N) + j * BN)
    take = rv < rowv_s[...]
    rowv_s[...] = jnp.where(take, rv, rowv_s[...])
    rowi_s[...] = jnp.where(take, ri, rowi_s[...])

    qbe = qn_s[pl.ds(j * BN, BN), :] + EPS
    dif = qbe - pn
    dm = jnp.sqrt(_row_sums(dif * dif))
    y = dm * dm
    for k in range(1, nneg + 1):
        dif = qbe - othersn[k]
        dm = jnp.sqrt(_row_sums(dif * dif))
        h = jnp.maximum(MARGIN - dm, 0.0)
        y = y + h * h
    y = y.reshape(BN)

    gidx = lax.iota(jnp.int32, BN) + j * BN
    ratio = jnp.sqrt(v1) / i2.astype(jnp.float32)
    colok = jnp.logical_and(i1 == gidx, ratio <= LOWE_RATIO_TH)

    coli1_out[j, :] = i1
    w_out[j, :] = jnp.where(colok, y, 0.0)

    @pl.when(j == NBLK - 1)
    def _final():
        rowi_out[...] = rowi_s[...]


def _sc_finalize_body(coli1_hbm, rowi_hbm, w_hbm, out_hbm,
                      rowi_v, coli1_v, w_v, acc_v):
    wid = lax.axis_index("s")
    r = wid // (BN // EPW)
    c0 = (wid % (BN // EPW)) * EPW
    base = r * BN + c0
    pltpu.sync_copy(rowi_hbm, rowi_v)
    pltpu.sync_copy(coli1_hbm.at[r, pl.ds(c0, EPW)], coli1_v)
    pltpu.sync_copy(w_hbm.at[r, pl.ds(c0, EPW)], w_v)
    acc_v[...] = jnp.zeros((VEC,), jnp.float32)
    zero = jnp.zeros((VEC,), jnp.int32)

    @pl.loop(0, EPW, step=VEC)
    def _(v):
        idx = coli1_v[pl.ds(v, VEC)]
        rg = plsc.load_gather(rowi_v, [zero, idx])
        jvec = lax.iota(jnp.int32, VEC) + (base + v)
        wv = w_v[pl.ds(v, VEC)]
        acc_v[...] = acc_v[...] + jnp.where(rg == jvec, wv, jnp.float32(0.0))

    pltpu.sync_copy(acc_v, out_hbm.at[wid])


@functools.cache
def _sc_finalize():
    mesh = plsc.VectorSubcoreMesh(
        core_axis_name="c", subcore_axis_name="s", num_cores=1)
    cp = pltpu.CompilerParams()
    if "needs_layout_passes" in pltpu.CompilerParams.__dataclass_fields__:
        cp = dataclasses.replace(cp, needs_layout_passes=False)
    return pl.kernel(
        _sc_finalize_body,
        out_type=jax.ShapeDtypeStruct((NW, VEC), jnp.float32),
        mesh=mesh,
        compiler_params=cp,
        scratch_types=[
            pltpu.VMEM((1, N), jnp.int32),
            pltpu.VMEM((EPW,), jnp.int32),
            pltpu.VMEM((EPW,), jnp.float32),
            pltpu.VMEM((VEC,), jnp.float32),
        ],
    )


@jax.jit
def kernel(superfeatures, target):
    del target
    nimg = superfeatures.shape[0]
    coli1, rowi, w = pl.pallas_call(
        functools.partial(_tc_kernel, nimg),
        grid=(NBLK,),
        in_specs=[
            pl.BlockSpec((1, N, D), lambda j: (0, 0, 0)),
            pl.BlockSpec((nimg, BN, D), lambda j: (0, j, 0)),
        ],
        out_specs=[
            pl.BlockSpec((NBLK, BN), lambda j: (0, 0)),
            pl.BlockSpec((1, N), lambda j: (0, 0)),
            pl.BlockSpec((NBLK, BN), lambda j: (0, 0)),
        ],
        out_shape=[
            jax.ShapeDtypeStruct((NBLK, BN), jnp.int32),
            jax.ShapeDtypeStruct((1, N), jnp.int32),
            jax.ShapeDtypeStruct((NBLK, BN), jnp.float32),
        ],
        scratch_shapes=[
            pltpu.VMEM((N, D), jnp.float32),
            pltpu.VMEM((N, D), jnp.float32),
            pltpu.VMEM((N, 1), jnp.float32),
            pltpu.VMEM((1, N), jnp.float32),
            pltpu.VMEM((1, N), jnp.float32),
            pltpu.VMEM((1, N), jnp.int32),
        ],
    )(superfeatures, superfeatures)
    partials = _sc_finalize()(coli1, rowi, w)
    return 0.5 * WEIGHT * jnp.sum(partials)
